# Initial kernel scaffold; baseline (speedup 1.0000x reference)
import jax
import jax.numpy as jnp
from jax.experimental import pallas as pl


def kernel(x_demand, x_measurement, edge_index, edge_label_index, edge_weight, W_rel1, W_root1_m, b1_m, W_root1_d, b1_d, W_rel2, W_root2_m, b2_m, W_root2_d, b2_d, W_dec1, b_dec1, W_dec2, b_dec2):
    raise NotImplementedError("write your pallas kernel here")


# speedup vs baseline: 3.7203x; 3.7203x over previous
import functools

import jax
import jax.numpy as jnp
from jax import lax
from jax.experimental import pallas as pl
from jax.experimental.pallas import tpu as pltpu
from jax.experimental.pallas import tpu_sc as plsc

N = 10000
E = 320000
D = 128
H = 128
O = 64
L = 40000
WP = H + O
WH = WP // 2

NC = 2
NS = 16

_sc_mesh = plsc.VectorSubcoreMesh(
    core_axis_name="c", subcore_axis_name="s", num_cores=NC, num_subcores=NS)

RB = 1000


def _dense1_body(xd_ref, xm_ref, wr1d_ref, b1d_ref, wrel1_ref, wrel2_ref,
                 wr1m_ref, hd_ref, pa_ref, pb_ref, r1m_ref):
    xd = xd_ref[...]
    hd = jnp.maximum(
        jnp.dot(xd, wr1d_ref[...], preferred_element_type=jnp.float32)
        + b1d_ref[...], 0.0)
    hd_ref[...] = hd
    p1 = jnp.dot(xd, wrel1_ref[...], preferred_element_type=jnp.float32)
    p2 = jnp.dot(hd, wrel2_ref[...], preferred_element_type=jnp.float32)
    pa_ref[...] = p1[:, :WH]
    pb_ref[...] = jnp.concatenate([p1[:, WH:], p2], axis=1)
    r1m_ref[...] = jnp.dot(xm_ref[...], wr1m_ref[...],
                           preferred_element_type=jnp.float32)


def _dense1(xd, xm, wr1d, b1d, wrel1, wrel2, wr1m):
    return pl.pallas_call(
        _dense1_body,
        grid=(N // RB,),
        in_specs=[
            pl.BlockSpec((RB, D), lambda i: (i, 0)),
            pl.BlockSpec((RB, D), lambda i: (i, 0)),
            pl.BlockSpec((D, H), lambda i: (0, 0)),
            pl.BlockSpec((1, H), lambda i: (0, 0)),
            pl.BlockSpec((D, H), lambda i: (0, 0)),
            pl.BlockSpec((H, O), lambda i: (0, 0)),
            pl.BlockSpec((D, H), lambda i: (0, 0)),
        ],
        out_specs=[
            pl.BlockSpec((RB, H), lambda i: (i, 0)),
            pl.BlockSpec((RB, WH), lambda i: (i, 0)),
            pl.BlockSpec((RB, WH), lambda i: (i, 0)),
            pl.BlockSpec((RB, H), lambda i: (i, 0)),
        ],
        out_shape=[
            jax.ShapeDtypeStruct((N, H), jnp.float32),
            jax.ShapeDtypeStruct((N, WH), jnp.float32),
            jax.ShapeDtypeStruct((N, WH), jnp.float32),
            jax.ShapeDtypeStruct((N, H), jnp.float32),
        ],
    )(xd, xm, wr1d, b1d, wrel1, wrel2, wr1m)


CH = 80
EPT = E // NS
NCH = EPT // CH
NVR = WH // 16


@functools.partial(
    pl.kernel,
    out_type=(jax.ShapeDtypeStruct((N, WH), jnp.float32),
              jax.ShapeDtypeStruct((N, WH), jnp.float32)),
    mesh=_sc_mesh,
    scratch_types=[
        pltpu.VMEM((CH,), jnp.int32),
        pltpu.VMEM((CH,), jnp.int32),
        pltpu.VMEM((CH,), jnp.float32),
        pltpu.VMEM((CH, WH), jnp.float32),
        pltpu.VMEM_SHARED((N, WH), jnp.float32),
        pltpu.SemaphoreType.DMA,
    ],
    compiler_params=pltpu.CompilerParams(use_tc_tiling_on_sc=False),
)
def _segsum(pa_hbm, pb_hbm, src_hbm, dst_hbm, w_hbm, out0_hbm, out1_hbm,
            src_v, dst_v, w_v, rows_v, acc_sh, sem):
    c = lax.axis_index("c")
    s = lax.axis_index("s")

    zero = jnp.zeros((16,), jnp.float32)

    def _zrow(e, carry):
        for r in range(NVR):
            rows_v[e, pl.ds(r * 16, 16)] = zero
        return carry

    lax.fori_loop(0, CH, _zrow, 0)

    def _zacc(i, carry):
        cid = s + i * NS

        @pl.when(cid < N // CH)
        def _():
            pltpu.sync_copy(rows_v, acc_sh.at[pl.ds(cid * CH, CH)])

        return carry

    lax.fori_loop(0, (N // CH + NS - 1) // NS, _zacc, 0)
    plsc.subcore_barrier()

    ebase = s * EPT

    def _make_pass(table_hbm):
        def _chunk(j, carry):
            eb = ebase + j * CH
            pltpu.sync_copy(src_hbm.at[pl.ds(eb, CH)], src_v)
            pltpu.sync_copy(dst_hbm.at[pl.ds(eb, CH)], dst_v)
            pltpu.sync_copy(w_hbm.at[pl.ds(eb, CH)], w_v)
            pltpu.async_copy(table_hbm.at[src_v], rows_v, sem).wait()

            def _grp(g, gcarry):
                wv16 = w_v[pl.ds(g * 16, 16)]
                for lane in range(16):
                    e = g * 16 + lane
                    wb = jnp.full((16,), wv16[lane], jnp.float32)
                    for r in range(NVR):
                        rows_v[e, pl.ds(r * 16, 16)] = (
                            rows_v[e, pl.ds(r * 16, 16)] * wb)
                return gcarry

            lax.fori_loop(0, CH // 16, _grp, 0)
            pltpu.sync_copy(rows_v, acc_sh.at[dst_v], add=True)
            return carry

        lax.fori_loop(0, NCH, _chunk, 0)

    @pl.when(c == 0)
    def _():
        _make_pass(pa_hbm)

    @pl.when(c == 1)
    def _():
        _make_pass(pb_hbm)

    plsc.subcore_barrier()

    def _copyout(i, carry):
        cid = s + i * NS

        @pl.when(cid < N // CH)
        def _():
            r0 = cid * CH

            @pl.when(c == 0)
            def _():
                pltpu.sync_copy(acc_sh.at[pl.ds(r0, CH)],
                                out0_hbm.at[pl.ds(r0, CH)])

            @pl.when(c == 1)
            def _():
                pltpu.sync_copy(acc_sh.at[pl.ds(r0, CH)],
                                out1_hbm.at[pl.ds(r0, CH)])

        return carry

    lax.fori_loop(0, (N // CH + NS - 1) // NS, _copyout, 0)


def _dense2_body(a0_ref, a1_ref, r1m_ref, hd_ref, wr2m_ref, b2m_ref,
                 wr2d_ref, b2d_ref, wd1_ref, b1m_ref, ud_ref, um_ref):
    a = jnp.concatenate([a0_ref[...], a1_ref[...]], axis=1)
    h_m = jnp.maximum(a[:, :H] + r1m_ref[...] + b1m_ref[...], 0.0)
    z_m = (a[:, H:]
           + jnp.dot(h_m, wr2m_ref[...], preferred_element_type=jnp.float32)
           + b2m_ref[...])
    z_d = (jnp.dot(hd_ref[...], wr2d_ref[...],
                   preferred_element_type=jnp.float32) + b2d_ref[...])
    wd1 = wd1_ref[...]
    ud_ref[...] = jnp.dot(z_d, wd1[:O], preferred_element_type=jnp.float32)
    um_ref[...] = jnp.dot(z_m, wd1[O:], preferred_element_type=jnp.float32)


def _dense2(a0, a1, r1m, hd, wr2m, b2m, wr2d, b2d, wd1, b1m):
    return pl.pallas_call(
        _dense2_body,
        grid=(N // RB,),
        in_specs=[
            pl.BlockSpec((RB, WH), lambda i: (i, 0)),
            pl.BlockSpec((RB, WH), lambda i: (i, 0)),
            pl.BlockSpec((RB, H), lambda i: (i, 0)),
            pl.BlockSpec((RB, H), lambda i: (i, 0)),
            pl.BlockSpec((H, O), lambda i: (0, 0)),
            pl.BlockSpec((1, O), lambda i: (0, 0)),
            pl.BlockSpec((H, O), lambda i: (0, 0)),
            pl.BlockSpec((1, O), lambda i: (0, 0)),
            pl.BlockSpec((2 * O, O), lambda i: (0, 0)),
            pl.BlockSpec((1, H), lambda i: (0, 0)),
        ],
        out_specs=[
            pl.BlockSpec((RB, O), lambda i: (i, 0)),
            pl.BlockSpec((RB, O), lambda i: (i, 0)),
        ],
        out_shape=[
            jax.ShapeDtypeStruct((N, O), jnp.float32),
            jax.ShapeDtypeStruct((N, O), jnp.float32),
        ],
    )(a0, a1, r1m, hd, wr2m, b2m, wr2d, b2d, wd1, b1m)


GCH = 80
TPT = 1280
GNCH = TPT // GCH


@functools.partial(
    pl.kernel,
    out_type=(jax.ShapeDtypeStruct((L, O), jnp.float32),
              jax.ShapeDtypeStruct((L, O), jnp.float32)),
    mesh=_sc_mesh,
    scratch_types=[
        pltpu.VMEM((GCH,), jnp.int32),
        pltpu.VMEM((GCH, O), jnp.float32),
        pltpu.SemaphoreType.DMA,
    ],
    compiler_params=pltpu.CompilerParams(use_tc_tiling_on_sc=False),
)
def _gather2(ud_hbm, um_hbm, row_hbm, col_hbm, gd_hbm, gm_hbm,
             idx_v, rows_v, sem):
    c = lax.axis_index("c")
    s = lax.axis_index("s")
    t = s * NC + c

    def _chunk(j, carry):
        b = t * TPT + j * GCH

        @pl.when(b < L)
        def _():
            pltpu.sync_copy(row_hbm.at[pl.ds(b, GCH)], idx_v)
            pltpu.async_copy(ud_hbm.at[idx_v], rows_v, sem).wait()
            pltpu.sync_copy(rows_v, gd_hbm.at[pl.ds(b, GCH)])
            pltpu.sync_copy(col_hbm.at[pl.ds(b, GCH)], idx_v)
            pltpu.async_copy(um_hbm.at[idx_v], rows_v, sem).wait()
            pltpu.sync_copy(rows_v, gm_hbm.at[pl.ds(b, GCH)])

        return carry

    lax.fori_loop(0, GNCH, _chunk, 0)


LB = 2000


def _dec_body(gd_ref, gm_ref, bd1_ref, w2_ref, bd2_ref, out_ref):
    h = jnp.maximum(gd_ref[...] + gm_ref[...] + bd1_ref[...], 0.0)
    logit = jnp.sum(h * w2_ref[...], axis=1, keepdims=True) + bd2_ref[...]
    out_ref[...] = jax.nn.sigmoid(logit)


def _dec(gd, gm, bd1, w2, bd2):
    return pl.pallas_call(
        _dec_body,
        grid=(L // LB,),
        in_specs=[
            pl.BlockSpec((LB, O), lambda i: (i, 0)),
            pl.BlockSpec((LB, O), lambda i: (i, 0)),
            pl.BlockSpec((1, O), lambda i: (0, 0)),
            pl.BlockSpec((1, O), lambda i: (0, 0)),
            pl.BlockSpec((1, 1), lambda i: (0, 0)),
        ],
        out_specs=pl.BlockSpec((LB, 1), lambda i: (i, 0)),
        out_shape=jax.ShapeDtypeStruct((L, 1), jnp.float32),
    )(gd, gm, bd1, w2, bd2)


def kernel(x_demand, x_measurement, edge_index, edge_label_index, edge_weight,
           W_rel1, W_root1_m, b1_m, W_root1_d, b1_d,
           W_rel2, W_root2_m, b2_m, W_root2_d, b2_d,
           W_dec1, b_dec1, W_dec2, b_dec2):
    hd, pa, pb, r1m = _dense1(x_demand, x_measurement, W_root1_d,
                              b1_d.reshape(1, H), W_rel1, W_rel2, W_root1_m)
    a0, a1 = _segsum(pa, pb, edge_index[0], edge_index[1], edge_weight)
    ud, um = _dense2(a0, a1, r1m, hd, W_root2_m, b2_m.reshape(1, O),
                     W_root2_d, b2_d.reshape(1, O), W_dec1, b1_m.reshape(1, H))
    gd, gm = _gather2(ud, um, edge_label_index[0], edge_label_index[1])
    out = _dec(gd, gm, b_dec1.reshape(1, O), W_dec2.reshape(1, O),
               b_dec2.reshape(1, 1))
    return out.reshape(L)
